# baseline (speedup 1.0000x reference)
import jax
import jax.numpy as jnp
from jax import lax
from jax.experimental import pallas as pl
from jax.experimental.pallas import tpu as pltpu
from jax.experimental.pallas import tpu_sc as plsc

D = 32
LANE = 128
K = 8
CH = K * LANE
NIB = 4
NRB = 2
NC, NS = 2, 16
NW = NC * NS


def _gather_body(idx_hbm, tbl_hbm, out_hbm, idx_v, rows_v, isem, gsem, osem):
    wid = lax.axis_index("s") * NC + lax.axis_index("c")
    n_per_w = idx_hbm.shape[0] // NW
    base0 = wid * n_per_w
    nch = n_per_w // CH

    def fire_idx(g, ib):
        pltpu.async_copy(
            idx_hbm.at[pl.ds(base0 + g * CH, CH)], idx_v.at[ib], isem.at[ib]
        )

    def wait_idx(ib):
        pltpu.make_async_copy(
            idx_hbm.at[pl.ds(base0, CH)], idx_v.at[ib], isem.at[ib]
        ).wait()

    def fire_gathers(ib, rb):
        for j in range(K):
            pltpu.async_copy(
                tbl_hbm.at[idx_v.at[ib, pl.ds(j * LANE, LANE)]],
                rows_v.at[rb, pl.ds(j * LANE, LANE)],
                gsem.at[rb],
            )

    def drain_gathers(rb):
        for j in range(K):
            pltpu.make_async_copy(
                tbl_hbm.at[idx_v.at[0, pl.ds(0, LANE)]],
                rows_v.at[rb, pl.ds(j * LANE, LANE)],
                gsem.at[rb],
            ).wait()

    def fire_store(g, rb):
        pltpu.async_copy(
            rows_v.at[rb], out_hbm.at[pl.ds(base0 + g * CH, CH)], osem.at[rb]
        )

    def wait_store(rb):
        pltpu.make_async_copy(
            rows_v.at[rb], out_hbm.at[pl.ds(base0, CH)], osem.at[rb]
        ).wait()

    for g in range(NIB - 1):
        fire_idx(g, g)

    def outer(i, carry):
        for u in range(NIB):
            g = NIB * i + u
            ib = u
            rb = u % NRB

            @pl.when(g >= NRB)
            def _():
                wait_store(rb)

            wait_idx(ib)
            fire_gathers(ib, rb)

            @pl.when(g >= 1)
            def _():
                drain_gathers(1 - rb)
                fire_store(g - 1, 1 - rb)

            @pl.when(g + NIB - 1 < nch)
            def _():
                fire_idx(g + NIB - 1, (u + NIB - 1) % NIB)
        return carry

    lax.fori_loop(0, nch // NIB, outer, 0)
    last_rb = (nch - 1) % NRB
    drain_gathers(last_rb)
    fire_store(nch - 1, last_rb)
    for rb in range(NRB):
        wait_store(rb)


def kernel(indexes, index2vec_weight):
    n = indexes.shape[0]
    assert n % (NW * CH * NIB) == 0
    mesh = plsc.VectorSubcoreMesh(core_axis_name="c", subcore_axis_name="s")
    f = pl.kernel(
        _gather_body,
        out_type=jax.ShapeDtypeStruct((n, D), jnp.float32),
        mesh=mesh,
        scratch_types=[
            pltpu.VMEM((NIB, CH), jnp.int32),
            pltpu.VMEM((NRB, CH, D), jnp.float32),
            pltpu.SemaphoreType.DMA((NIB,)),
            pltpu.SemaphoreType.DMA((NRB,)),
            pltpu.SemaphoreType.DMA((NRB,)),
        ],
        compiler_params=pltpu.CompilerParams(use_tc_tiling_on_sc=False),
    )
    return f(indexes, index2vec_weight)
